# Initial kernel scaffold; baseline (speedup 1.0000x reference)
import jax
import jax.numpy as jnp
from jax.experimental import pallas as pl


def kernel(x, edge_index, batch, W1_root, W1_rel, b1, W2_root, W2_rel, b2, lin1_W, lin1_b, lin2_W, lin2_b):
    raise NotImplementedError("write your pallas kernel here")


# speedup vs baseline: 2.3123x; 2.3123x over previous
import functools

import jax
import jax.numpy as jnp
from jax import lax
from jax.experimental import pallas as pl
from jax.experimental.pallas import tpu as pltpu
from jax.experimental.pallas import tpu_sc as plsc

_NC = 2
_NS = 16
_C = 128
_NP = 102400
_RB = 3200
_FW = 8


def _make_sc_agg(epad, npp, split_by_core, scale):
  n_slabs = _NC * npp
  workers = _NC * _NS if split_by_core else _NS
  ept = epad // workers
  groups = ept // _C
  rpt = _NP // _NS
  nrb = rpt // _RB

  mesh = plsc.VectorSubcoreMesh(core_axis_name="c", subcore_axis_name="s")

  @functools.partial(
      pl.kernel,
      out_type=jax.ShapeDtypeStruct((n_slabs, _NP, _FW), jnp.float32),
      mesh=mesh,
      scratch_types=[
          pltpu.VMEM((_C,), jnp.int32),
          pltpu.VMEM((_C,), jnp.int32),
          pltpu.VMEM((_C,), jnp.int32),
          pltpu.VMEM((_C, _FW), jnp.float32),
          pltpu.VMEM((_RB, 16), jnp.float32),
          pltpu.VMEM_SHARED((_NP, _FW), jnp.float32),
          pltpu.SemaphoreType.DMA,
      ],
      compiler_params=pltpu.CompilerParams(use_tc_tiling_on_sc=False),
  )
  def k(table, src_h, dst_h, out, src_raw, src_idx, dst_idx, rows, zbuf,
        acc, sem):
    cid = lax.axis_index("c")
    sid = lax.axis_index("s")

    zero16 = jnp.zeros((16,), jnp.float32)

    def zfill(i, carry):
      zbuf[i] = zero16
      return carry

    lax.fori_loop(0, _RB, zfill, 0)

    for j in range(npp):
      slab = cid * npp + j

      def zrow(i, carry):
        pltpu.sync_copy(zbuf.at[:, pl.ds(0, _FW)],
                        acc.at[pl.ds(sid * rpt + i * _RB, _RB)])
        return carry

      lax.fori_loop(0, nrb, zrow, 0)
      plsc.subcore_barrier()

      if split_by_core:
        base = (cid * _NS + sid) * ept
      else:
        base = sid * ept

      def grp(g, carry):
        e0 = base + g * _C
        pltpu.sync_copy(src_h.at[pl.ds(e0, _C)], src_raw)
        pltpu.sync_copy(dst_h.at[pl.ds(e0, _C)], dst_idx)
        if scale == 1:
          gidx = src_raw
        else:
          for kk in range(_C // 16):
            sl = pl.ds(kk * 16, 16)
            src_idx[sl] = src_raw[sl] * scale + slab
          gidx = src_idx
        pltpu.async_copy(table.at[gidx], rows, sem).wait()
        pltpu.sync_copy(rows, acc.at[dst_idx], add=True)
        return carry

      lax.fori_loop(0, groups, grp, 0)
      plsc.subcore_barrier()

      def rback(i, carry):
        r0 = sid * rpt + i * _RB
        pltpu.sync_copy(acc.at[pl.ds(r0, _RB)], out.at[slab, pl.ds(r0, _RB)])
        return carry

      lax.fori_loop(0, nrb, rback, 0)

  return k


def _tc1_body(xr, ar, wr, wl, br, out):
  a = ar[0] + ar[1]
  z = jnp.dot(xr[...], wr[...], preferred_element_type=jnp.float32)
  z = z + jnp.dot(a, wl[...], preferred_element_type=jnp.float32)
  z = z + br[...]
  out[...] = jnp.maximum(z, 0.0)


def _make_tc2_body(n_grid, n_graphs):
  def body(h1r, a2r, btr, w2r, w2l, b2r, l1w, l1b, l2w, l2b, outr, sums,
           counts):
    i = pl.program_id(0)

    @pl.when(i == 0)
    def _():
      sums[...] = jnp.zeros_like(sums)
      counts[...] = jnp.zeros_like(counts)

    z = jnp.dot(h1r[...], w2r[...], preferred_element_type=jnp.float32)
    for p in range(64 // _FW):
      z += jnp.dot(a2r[p], w2l[p * _FW:(p + 1) * _FW, :],
                   preferred_element_type=jnp.float32)
    z = z + b2r[...]
    h2 = jnp.maximum(z, 0.0)

    bt = btr[0, 0, :]
    onehot = (bt[:, None] == lax.broadcasted_iota(
        jnp.int32, (1, n_graphs), 1)).astype(jnp.float32)
    sums[...] += lax.dot_general(onehot, h2, (((0,), (0,)), ((), ())),
                                 preferred_element_type=jnp.float32)
    counts[...] += jnp.sum(onehot, axis=0, keepdims=True)

    @pl.when(i == n_grid - 1)
    def _():
      cnt = jnp.maximum(counts[0, :], 1.0)
      pooled = sums[...] / cnt[:, None]
      h3 = jnp.maximum(
          jnp.dot(pooled, l1w[...], preferred_element_type=jnp.float32)
          + l1b[...], 0.0)
      outr[...] = jnp.dot(h3, l2w[...],
                          preferred_element_type=jnp.float32) + l2b[...]

  return body


def kernel(x, edge_index, batch, W1_root, W1_rel, b1, W2_root, W2_rel, b2,
           lin1_W, lin1_b, lin2_W, lin2_b):
  n, f = x.shape
  e = edge_index.shape[1]
  g = 128
  bn = 2048
  ng = _NP // bn

  unit = _NC * _NS * _C
  epad = ((e + unit - 1) // unit) * unit
  pad = epad - e
  src = jnp.concatenate([edge_index[0], jnp.zeros((pad,), jnp.int32)])
  dst = jnp.concatenate([edge_index[1], jnp.full((pad,), n, jnp.int32)])

  x8 = jnp.pad(x, ((0, _NP - n), (0, _FW - f)))
  w1r8 = jnp.pad(W1_root, ((0, _FW - f), (0, 0)))
  w1l8 = jnp.pad(W1_rel, ((0, _FW - f), (0, 0)))
  batch_p = jnp.concatenate([batch, jnp.full((_NP - n,), g, jnp.int32)])

  agg1p = _make_sc_agg(epad, npp=1, split_by_core=True, scale=1)(
      x8, src, dst)

  h1 = pl.pallas_call(
      _tc1_body,
      grid=(ng,),
      in_specs=[
          pl.BlockSpec((bn, _FW), lambda i: (i, 0)),
          pl.BlockSpec((2, bn, _FW), lambda i: (0, i, 0)),
          pl.BlockSpec((_FW, 64), lambda i: (0, 0)),
          pl.BlockSpec((_FW, 64), lambda i: (0, 0)),
          pl.BlockSpec((1, 64), lambda i: (0, 0)),
      ],
      out_specs=pl.BlockSpec((bn, 64), lambda i: (i, 0)),
      out_shape=jax.ShapeDtypeStruct((_NP, 64), jnp.float32),
  )(x8, agg1p, w1r8, w1l8, b1.reshape(1, 64))

  npp2 = (64 // _FW) // _NC
  agg2 = _make_sc_agg(epad, npp=npp2, split_by_core=False, scale=64 // _FW)(
      h1.reshape(_NP * (64 // _FW), _FW), src, dst)

  out = pl.pallas_call(
      _make_tc2_body(ng, g),
      grid=(ng,),
      in_specs=[
          pl.BlockSpec((bn, 64), lambda i: (i, 0)),
          pl.BlockSpec((64 // _FW, bn, _FW), lambda i: (0, i, 0)),
          pl.BlockSpec((1, 1, bn), lambda i: (i, 0, 0)),
          pl.BlockSpec((64, 64), lambda i: (0, 0)),
          pl.BlockSpec((64, 64), lambda i: (0, 0)),
          pl.BlockSpec((1, 64), lambda i: (0, 0)),
          pl.BlockSpec((64, 32), lambda i: (0, 0)),
          pl.BlockSpec((1, 32), lambda i: (0, 0)),
          pl.BlockSpec((32, 2), lambda i: (0, 0)),
          pl.BlockSpec((1, 2), lambda i: (0, 0)),
      ],
      out_specs=pl.BlockSpec((g, 2), lambda i: (0, 0)),
      out_shape=jax.ShapeDtypeStruct((g, 2), jnp.float32),
      scratch_shapes=[
          pltpu.VMEM((g, 64), jnp.float32),
          pltpu.VMEM((1, g), jnp.float32),
      ],
  )(h1, agg2, batch_p.reshape(ng, 1, bn), W2_root, W2_rel,
    b2.reshape(1, 64), lin1_W, lin1_b.reshape(1, 32), lin2_W,
    lin2_b.reshape(1, 2))

  return out


# speedup vs baseline: 7.5860x; 3.2807x over previous
import functools

import jax
import jax.numpy as jnp
from jax import lax
from jax.experimental import pallas as pl
from jax.experimental.pallas import tpu as pltpu
from jax.experimental.pallas import tpu_sc as plsc

_NC = 2
_NS = 16
_C = 128
_NP = 102400
_RB = 3200
_FW = 8
_NI = 8
_NR = 4


def _make_sc_agg(epad, npp, split_by_core, scale):
  n_slabs = _NC * npp
  workers = _NC * _NS if split_by_core else _NS
  ept = epad // workers
  groups = ept // _C
  assert groups % _NI == 0 and groups >= 2 * _NI
  touter = groups // _NI
  rpt = _NP // _NS
  nrb = rpt // _RB

  mesh = plsc.VectorSubcoreMesh(core_axis_name="c", subcore_axis_name="s")

  scratch = (
      [pltpu.VMEM((_C,), jnp.int32) for _ in range(_NI)]
      + [pltpu.VMEM((_C,), jnp.int32) for _ in range(_NI)]
      + [pltpu.VMEM((_C,), jnp.int32) for _ in range(_NI)]
      + [pltpu.VMEM((_C, _FW), jnp.float32) for _ in range(_NR)]
      + [
          pltpu.VMEM((_RB, 16), jnp.float32),
          pltpu.VMEM_SHARED((_NP, _FW), jnp.float32),
      ]
      + [pltpu.SemaphoreType.DMA for _ in range(_NI)]
      + [pltpu.SemaphoreType.DMA for _ in range(_NR)]
      + [pltpu.SemaphoreType.DMA for _ in range(_NR)]
  )

  @functools.partial(
      pl.kernel,
      out_type=jax.ShapeDtypeStruct((n_slabs, _NP, _FW), jnp.float32),
      mesh=mesh,
      scratch_types=scratch,
      compiler_params=pltpu.CompilerParams(use_tc_tiling_on_sc=False),
  )
  def k(table, src_h, dst_h, out, *scr):
    srcb = scr[0:_NI]
    gib = scr[_NI:2 * _NI]
    dstb = scr[2 * _NI:3 * _NI]
    rows = scr[3 * _NI:3 * _NI + _NR]
    zbuf = scr[3 * _NI + _NR]
    acc = scr[3 * _NI + _NR + 1]
    sem_i = scr[3 * _NI + _NR + 2:4 * _NI + _NR + 2]
    sem_g = scr[4 * _NI + _NR + 2:4 * _NI + 2 * _NR + 2]
    sem_s = scr[4 * _NI + 2 * _NR + 2:4 * _NI + 3 * _NR + 2]

    cid = lax.axis_index("c")
    sid = lax.axis_index("s")

    zero16 = jnp.zeros((16,), jnp.float32)

    def zfill(i, carry):
      zbuf[i] = zero16
      return carry

    lax.fori_loop(0, _RB, zfill, 0)

    for j in range(npp):
      slab = cid * npp + j

      def zrow(i, carry):
        pltpu.sync_copy(zbuf.at[:, pl.ds(0, _FW)],
                        acc.at[pl.ds(sid * rpt + i * _RB, _RB)])
        return carry

      lax.fori_loop(0, nrb, zrow, 0)
      plsc.subcore_barrier()

      if split_by_core:
        base = (cid * _NS + sid) * ept
      else:
        base = sid * ept

      def start_idx(g, b):
        e0 = base + g * _C
        pltpu.async_copy(src_h.at[pl.ds(e0, _C)], srcb[b], sem_i[b])
        pltpu.async_copy(dst_h.at[pl.ds(e0, _C)], dstb[b], sem_i[b])

      def wait_idx(b):
        pltpu.make_async_copy(src_h.at[pl.ds(0, _C)], srcb[b],
                              sem_i[b]).wait()
        pltpu.make_async_copy(dst_h.at[pl.ds(0, _C)], dstb[b],
                              sem_i[b]).wait()

      def gidx_ref(b):
        if scale == 1:
          return srcb[b]
        for kk in range(_C // 16):
          sl = pl.ds(kk * 16, 16)
          gib[b][sl] = srcb[b][sl] * scale + slab
        return gib[b]

      def start_gather(b8, b4):
        pltpu.async_copy(table.at[gidx_ref(b8)], rows[b4], sem_g[b4])

      def wait_gather(b8, b4):
        ref = srcb[b8] if scale == 1 else gib[b8]
        pltpu.make_async_copy(table.at[ref], rows[b4], sem_g[b4]).wait()

      def start_scatter(b8, b4):
        pltpu.async_copy(rows[b4], acc.at[dstb[b8]], sem_s[b4], add=True)

      def wait_scatter(b8, b4):
        pltpu.make_async_copy(rows[b4], acc.at[dstb[b8]],
                              sem_s[b4]).wait()

      for b in range(4):
        start_idx(b, b)
      for b in range(2):
        wait_idx(b)
        start_gather(b, b)

      def body(h, first, last):
        for b in range(_NI):
          g = h + b
          b4 = b % _NR
          b8 = b
          a_b4 = (b + 2) % _NR
          a_b8 = (b + 2) % _NI
          do_a = (not last) or (b < _NI - 2)
          if do_a:
            if not (first and b < 2):
              wait_scatter((b8 - 2) % _NI, a_b4)
            wait_idx(a_b8)
            start_gather(a_b8, a_b4)
          wait_gather(b8, b4)
          start_scatter(b8, b4)
          if (not last) or (b < _NI - 4):
            start_idx(g + 4, (b + 4) % _NI)

      body(0, True, False)
      if touter > 2:
        def mid(t, carry):
          body(t * _NI, False, False)
          return carry
        lax.fori_loop(1, touter - 1, mid, 0)
      body((touter - 1) * _NI, False, True)

      for b in range(_NI - 4, _NI):
        wait_scatter(b % _NI, b % _NR)

      plsc.subcore_barrier()

      def rback(i, carry):
        r0 = sid * rpt + i * _RB
        pltpu.sync_copy(acc.at[pl.ds(r0, _RB)], out.at[slab, pl.ds(r0, _RB)])
        return carry

      lax.fori_loop(0, nrb, rback, 0)

  return k


def _tc1_body(xr, ar, wr, wl, br, out):
  a = ar[0] + ar[1]
  z = jnp.dot(xr[...], wr[...], preferred_element_type=jnp.float32)
  z = z + jnp.dot(a, wl[...], preferred_element_type=jnp.float32)
  z = z + br[...]
  out[...] = jnp.maximum(z, 0.0)


def _make_tc2_body(n_grid, n_graphs):
  def body(h1r, a2r, btr, w2r, w2l, b2r, l1w, l1b, l2w, l2b, outr, sums,
           counts):
    i = pl.program_id(0)

    @pl.when(i == 0)
    def _():
      sums[...] = jnp.zeros_like(sums)
      counts[...] = jnp.zeros_like(counts)

    z = jnp.dot(h1r[...], w2r[...], preferred_element_type=jnp.float32)
    for p in range(64 // _FW):
      z += jnp.dot(a2r[p], w2l[p * _FW:(p + 1) * _FW, :],
                   preferred_element_type=jnp.float32)
    z = z + b2r[...]
    h2 = jnp.maximum(z, 0.0)

    bt = btr[0, 0, :]
    onehot = (bt[:, None] == lax.broadcasted_iota(
        jnp.int32, (1, n_graphs), 1)).astype(jnp.float32)
    sums[...] += lax.dot_general(onehot, h2, (((0,), (0,)), ((), ())),
                                 preferred_element_type=jnp.float32)
    counts[...] += jnp.sum(onehot, axis=0, keepdims=True)

    @pl.when(i == n_grid - 1)
    def _():
      cnt = jnp.maximum(counts[0, :], 1.0)
      pooled = sums[...] / cnt[:, None]
      h3 = jnp.maximum(
          jnp.dot(pooled, l1w[...], preferred_element_type=jnp.float32)
          + l1b[...], 0.0)
      outr[...] = jnp.dot(h3, l2w[...],
                          preferred_element_type=jnp.float32) + l2b[...]

  return body


def kernel(x, edge_index, batch, W1_root, W1_rel, b1, W2_root, W2_rel, b2,
           lin1_W, lin1_b, lin2_W, lin2_b):
  n, f = x.shape
  e = edge_index.shape[1]
  g = 128
  bn = 2048
  ng = _NP // bn

  unit = _NC * _NS * _C * _NI
  epad = ((e + unit - 1) // unit) * unit
  pad = epad - e
  src = jnp.concatenate([edge_index[0], jnp.zeros((pad,), jnp.int32)])
  dst = jnp.concatenate([edge_index[1], jnp.full((pad,), n, jnp.int32)])

  x8 = jnp.pad(x, ((0, _NP - n), (0, _FW - f)))
  w1r8 = jnp.pad(W1_root, ((0, _FW - f), (0, 0)))
  w1l8 = jnp.pad(W1_rel, ((0, _FW - f), (0, 0)))
  batch_p = jnp.concatenate([batch, jnp.full((_NP - n,), g, jnp.int32)])

  agg1p = _make_sc_agg(epad, npp=1, split_by_core=True, scale=1)(
      x8, src, dst)

  h1 = pl.pallas_call(
      _tc1_body,
      grid=(ng,),
      in_specs=[
          pl.BlockSpec((bn, _FW), lambda i: (i, 0)),
          pl.BlockSpec((2, bn, _FW), lambda i: (0, i, 0)),
          pl.BlockSpec((_FW, 64), lambda i: (0, 0)),
          pl.BlockSpec((_FW, 64), lambda i: (0, 0)),
          pl.BlockSpec((1, 64), lambda i: (0, 0)),
      ],
      out_specs=pl.BlockSpec((bn, 64), lambda i: (i, 0)),
      out_shape=jax.ShapeDtypeStruct((_NP, 64), jnp.float32),
  )(x8, agg1p, w1r8, w1l8, b1.reshape(1, 64))

  npp2 = (64 // _FW) // _NC
  agg2 = _make_sc_agg(epad, npp=npp2, split_by_core=False, scale=64 // _FW)(
      h1.reshape(_NP * (64 // _FW), _FW), src, dst)

  out = pl.pallas_call(
      _make_tc2_body(ng, g),
      grid=(ng,),
      in_specs=[
          pl.BlockSpec((bn, 64), lambda i: (i, 0)),
          pl.BlockSpec((64 // _FW, bn, _FW), lambda i: (0, i, 0)),
          pl.BlockSpec((1, 1, bn), lambda i: (i, 0, 0)),
          pl.BlockSpec((64, 64), lambda i: (0, 0)),
          pl.BlockSpec((64, 64), lambda i: (0, 0)),
          pl.BlockSpec((1, 64), lambda i: (0, 0)),
          pl.BlockSpec((64, 32), lambda i: (0, 0)),
          pl.BlockSpec((1, 32), lambda i: (0, 0)),
          pl.BlockSpec((32, 2), lambda i: (0, 0)),
          pl.BlockSpec((1, 2), lambda i: (0, 0)),
      ],
      out_specs=pl.BlockSpec((g, 2), lambda i: (0, 0)),
      out_shape=jax.ShapeDtypeStruct((g, 2), jnp.float32),
      scratch_shapes=[
          pltpu.VMEM((g, 64), jnp.float32),
          pltpu.VMEM((1, g), jnp.float32),
      ],
  )(h1, agg2, batch_p.reshape(ng, 1, bn), W2_root, W2_rel,
    b2.reshape(1, 64), lin1_W, lin1_b.reshape(1, 32), lin2_W,
    lin2_b.reshape(1, 2))

  return out


# speedup vs baseline: 8.1697x; 1.0770x over previous
import functools

import jax
import jax.numpy as jnp
from jax import lax
from jax.experimental import pallas as pl
from jax.experimental.pallas import tpu as pltpu
from jax.experimental.pallas import tpu_sc as plsc

_NC = 2
_NS = 16
_C = 128
_NP = 102400
_RB = 3200
_FW = 8
_NI = 8
_NR = 4


def _make_sc_agg(epad, npp, split_by_core, scale):
  n_slabs = _NC * npp
  workers = _NC * _NS if split_by_core else _NS
  ept = epad // workers
  groups = ept // _C
  assert groups % _NI == 0 and groups >= 4 * _NI
  touter = groups // _NI
  rpt = _NP // _NS
  nrb = rpt // _RB

  mesh = plsc.VectorSubcoreMesh(core_axis_name="c", subcore_axis_name="s")

  scratch = (
      [pltpu.VMEM((_NI, _C), jnp.int32) for _ in range(3)]
      + [pltpu.VMEM((_NI, _C), jnp.int32) for _ in range(3)]
      + [pltpu.VMEM((_C,), jnp.int32) for _ in range(_NI)]
      + [pltpu.VMEM((_C, _FW), jnp.float32) for _ in range(_NR)]
      + [
          pltpu.VMEM((_RB, 16), jnp.float32),
          pltpu.VMEM_SHARED((_NP, _FW), jnp.float32),
      ]
      + [pltpu.SemaphoreType.DMA for _ in range(3)]
      + [pltpu.SemaphoreType.DMA for _ in range(_NR)]
      + [pltpu.SemaphoreType.DMA for _ in range(_NR)]
  )

  @functools.partial(
      pl.kernel,
      out_type=jax.ShapeDtypeStruct((n_slabs, _NP, _FW), jnp.float32),
      mesh=mesh,
      scratch_types=scratch,
      compiler_params=pltpu.CompilerParams(use_tc_tiling_on_sc=False),
  )
  def k(table, src_h, dst_h, out, *scr):
    srcb = scr[0:3]
    dstb = scr[3:6]
    gib = scr[6:6 + _NI]
    rows = scr[6 + _NI:6 + _NI + _NR]
    zbuf = scr[6 + _NI + _NR]
    acc = scr[6 + _NI + _NR + 1]
    sem_i = scr[6 + _NI + _NR + 2:6 + _NI + _NR + 5]
    sem_g = scr[6 + _NI + _NR + 5:6 + _NI + 2 * _NR + 5]
    sem_s = scr[6 + _NI + 2 * _NR + 5:6 + _NI + 3 * _NR + 5]

    cid = lax.axis_index("c")
    sid = lax.axis_index("s")

    zero16 = jnp.zeros((16,), jnp.float32)

    def zfill(i, carry):
      zbuf[i] = zero16
      return carry

    lax.fori_loop(0, _RB, zfill, 0)

    for j in range(npp):
      slab = cid * npp + j

      def zrow(i, carry):
        pltpu.sync_copy(zbuf.at[:, pl.ds(0, _FW)],
                        acc.at[pl.ds(sid * rpt + i * _RB, _RB)])
        return carry

      lax.fori_loop(0, nrb, zrow, 0)
      plsc.subcore_barrier()

      if split_by_core:
        base_row = (cid * _NS + sid) * groups
      else:
        base_row = sid * groups

      def start_idx_block(t, bi):
        r0 = base_row + t * _NI
        pltpu.async_copy(src_h.at[pl.ds(r0, _NI)], srcb[bi], sem_i[bi])
        pltpu.async_copy(dst_h.at[pl.ds(r0, _NI)], dstb[bi], sem_i[bi])

      def wait_idx_block(bi):
        pltpu.make_async_copy(src_h.at[pl.ds(0, _NI)], srcb[bi],
                              sem_i[bi]).wait()
        pltpu.make_async_copy(dst_h.at[pl.ds(0, _NI)], dstb[bi],
                              sem_i[bi]).wait()

      def gidx_ref(bi, row, b8):
        if scale == 1:
          return srcb[bi].at[row]
        for kk in range(_C // 16):
          sl = pl.ds(kk * 16, 16)
          gib[b8][sl] = srcb[bi][row, sl] * scale + slab
        return gib[b8]

      def start_gather(bi, row, b8, b4):
        pltpu.async_copy(table.at[gidx_ref(bi, row, b8)], rows[b4],
                         sem_g[b4])

      def wait_gather(b4):
        pltpu.make_async_copy(table.at[dstb[0].at[0]], rows[b4],
                              sem_g[b4]).wait()

      def start_scatter(bi, row, b4):
        pltpu.async_copy(rows[b4], acc.at[dstb[bi].at[row]], sem_s[b4],
                         add=True)

      def wait_scatter(b4):
        pltpu.make_async_copy(rows[b4], acc.at[dstb[0].at[0]],
                              sem_s[b4]).wait()

      start_idx_block(0, 0)
      start_idx_block(1, 1)
      wait_idx_block(0)
      for b in range(2):
        start_gather(0, b, b, b)

      def body(t, bi_cur, bi_nxt, kind):
        for b in range(_NI):
          b4 = b % _NR
          a_b4 = (b + 2) % _NR
          a_b8 = (b + 2) % _NI
          do_a = (kind != 'last') or (b < _NI - 2)
          if do_a:
            if not (kind == 'first' and b < 2):
              wait_scatter(a_b4)
            if b == 6 and kind != 'last':
              wait_idx_block(bi_nxt)
            if b + 2 < _NI:
              start_gather(bi_cur, b + 2, a_b8, a_b4)
            else:
              start_gather(bi_nxt, b + 2 - _NI, a_b8, a_b4)
          wait_gather(b4)
          start_scatter(bi_cur, b, b4)
          if b == 2 and kind in ('first', 'mid'):
            start_idx_block(t + 2, (bi_cur + 2) % 3)

      body(0, 0, 1, 'first')
      n_mid = touter - 3
      m3 = n_mid // 3
      if m3 > 0:
        def mid(s, carry):
          ts = 1 + 3 * s
          for d in range(3):
            bi = (1 + d) % 3
            body(ts + d, bi, (bi + 1) % 3, 'mid')
          return carry
        lax.fori_loop(0, m3, mid, 0)
      for t in range(1 + 3 * m3, touter - 2):
        body(t, t % 3, (t + 1) % 3, 'mid')
      body(touter - 2, (touter - 2) % 3, (touter - 1) % 3, 'pen')
      body(touter - 1, (touter - 1) % 3, touter % 3, 'last')

      for b in range(_NR):
        wait_scatter(b)

      plsc.subcore_barrier()

      def rback(i, carry):
        r0 = sid * rpt + i * _RB
        pltpu.sync_copy(acc.at[pl.ds(r0, _RB)], out.at[slab, pl.ds(r0, _RB)])
        return carry

      lax.fori_loop(0, nrb, rback, 0)

  return k


def _tc1_body(xr, ar, wr, wl, br, out):
  a = ar[0] + ar[1]
  z = jnp.dot(xr[...], wr[...], preferred_element_type=jnp.float32)
  z = z + jnp.dot(a, wl[...], preferred_element_type=jnp.float32)
  z = z + br[...]
  out[...] = jnp.maximum(z, 0.0)


def _make_tc2_body(n_grid, n_graphs):
  def body(h1r, a2r, btr, w2r, w2l, b2r, l1w, l1b, l2w, l2b, outr, sums,
           counts):
    i = pl.program_id(0)

    @pl.when(i == 0)
    def _():
      sums[...] = jnp.zeros_like(sums)
      counts[...] = jnp.zeros_like(counts)

    z = jnp.dot(h1r[...], w2r[...], preferred_element_type=jnp.float32)
    for p in range(64 // _FW):
      z += jnp.dot(a2r[p], w2l[p * _FW:(p + 1) * _FW, :],
                   preferred_element_type=jnp.float32)
    z = z + b2r[...]
    h2 = jnp.maximum(z, 0.0)

    bt = btr[0, 0, :]
    onehot = (bt[:, None] == lax.broadcasted_iota(
        jnp.int32, (1, n_graphs), 1)).astype(jnp.float32)
    sums[...] += lax.dot_general(onehot, h2, (((0,), (0,)), ((), ())),
                                 preferred_element_type=jnp.float32)
    counts[...] += jnp.sum(onehot, axis=0, keepdims=True)

    @pl.when(i == n_grid - 1)
    def _():
      cnt = jnp.maximum(counts[0, :], 1.0)
      pooled = sums[...] / cnt[:, None]
      h3 = jnp.maximum(
          jnp.dot(pooled, l1w[...], preferred_element_type=jnp.float32)
          + l1b[...], 0.0)
      outr[...] = jnp.dot(h3, l2w[...],
                          preferred_element_type=jnp.float32) + l2b[...]

  return body


def kernel(x, edge_index, batch, W1_root, W1_rel, b1, W2_root, W2_rel, b2,
           lin1_W, lin1_b, lin2_W, lin2_b):
  n, f = x.shape
  e = edge_index.shape[1]
  g = 128
  bn = 2048
  ng = _NP // bn

  unit = _NC * _NS * _C * _NI
  epad = ((e + unit - 1) // unit) * unit
  pad = epad - e
  src = jnp.concatenate([edge_index[0],
                         jnp.zeros((pad,), jnp.int32)]).reshape(-1, _C)
  dst = jnp.concatenate([edge_index[1],
                         jnp.full((pad,), n, jnp.int32)]).reshape(-1, _C)

  x8 = jnp.pad(x, ((0, _NP - n), (0, _FW - f)))
  w1r8 = jnp.pad(W1_root, ((0, _FW - f), (0, 0)))
  w1l8 = jnp.pad(W1_rel, ((0, _FW - f), (0, 0)))
  batch_p = jnp.concatenate([batch, jnp.full((_NP - n,), g, jnp.int32)])

  agg1p = _make_sc_agg(epad, npp=1, split_by_core=True, scale=1)(
      x8, src, dst)

  h1 = pl.pallas_call(
      _tc1_body,
      grid=(ng,),
      in_specs=[
          pl.BlockSpec((bn, _FW), lambda i: (i, 0)),
          pl.BlockSpec((2, bn, _FW), lambda i: (0, i, 0)),
          pl.BlockSpec((_FW, 64), lambda i: (0, 0)),
          pl.BlockSpec((_FW, 64), lambda i: (0, 0)),
          pl.BlockSpec((1, 64), lambda i: (0, 0)),
      ],
      out_specs=pl.BlockSpec((bn, 64), lambda i: (i, 0)),
      out_shape=jax.ShapeDtypeStruct((_NP, 64), jnp.float32),
  )(x8, agg1p, w1r8, w1l8, b1.reshape(1, 64))

  npp2 = (64 // _FW) // _NC
  agg2 = _make_sc_agg(epad, npp=npp2, split_by_core=False, scale=64 // _FW)(
      h1.reshape(_NP * (64 // _FW), _FW), src, dst)

  out = pl.pallas_call(
      _make_tc2_body(ng, g),
      grid=(ng,),
      in_specs=[
          pl.BlockSpec((bn, 64), lambda i: (i, 0)),
          pl.BlockSpec((64 // _FW, bn, _FW), lambda i: (0, i, 0)),
          pl.BlockSpec((1, 1, bn), lambda i: (i, 0, 0)),
          pl.BlockSpec((64, 64), lambda i: (0, 0)),
          pl.BlockSpec((64, 64), lambda i: (0, 0)),
          pl.BlockSpec((1, 64), lambda i: (0, 0)),
          pl.BlockSpec((64, 32), lambda i: (0, 0)),
          pl.BlockSpec((1, 32), lambda i: (0, 0)),
          pl.BlockSpec((32, 2), lambda i: (0, 0)),
          pl.BlockSpec((1, 2), lambda i: (0, 0)),
      ],
      out_specs=pl.BlockSpec((g, 2), lambda i: (0, 0)),
      out_shape=jax.ShapeDtypeStruct((g, 2), jnp.float32),
      scratch_shapes=[
          pltpu.VMEM((g, 64), jnp.float32),
          pltpu.VMEM((1, g), jnp.float32),
      ],
  )(h1, agg2, batch_p.reshape(ng, 1, bn), W2_root, W2_rel,
    b2.reshape(1, 64), lin1_W, lin1_b.reshape(1, 32), lin2_W,
    lin2_b.reshape(1, 2))

  return out


# speedup vs baseline: 10.3431x; 1.2660x over previous
import functools

import jax
import jax.numpy as jnp
from jax import lax
from jax.experimental import pallas as pl
from jax.experimental.pallas import tpu as pltpu
from jax.experimental.pallas import tpu_sc as plsc

_NC = 2
_NS = 16
_C = 128
_NP = 102400
_RB = 3200
_FW = 8
_NI = 8
_NR = 4


def _make_sc_agg(epad, npp, split_by_core, scale):
  n_slabs = _NC * npp
  workers = _NC * _NS if split_by_core else _NS
  ept = epad // workers
  groups = ept // _C
  assert groups % _NI == 0 and groups >= 4 * _NI
  touter = groups // _NI
  rpt = _NP // _NS
  nrb = rpt // _RB

  mesh = plsc.VectorSubcoreMesh(core_axis_name="c", subcore_axis_name="s")

  scratch = (
      [pltpu.VMEM((_NI, _C), jnp.int32) for _ in range(3)]
      + [pltpu.VMEM((_NI, _C), jnp.int32) for _ in range(3)]
      + [pltpu.VMEM((_C,), jnp.int32) for _ in range(_NI)]
      + [pltpu.VMEM((_C, _FW), jnp.float32) for _ in range(_NR)]
      + [
          pltpu.VMEM((_RB, 16), jnp.float32),
          pltpu.VMEM_SHARED((_NP, _FW), jnp.float32),
      ]
      + [pltpu.SemaphoreType.DMA for _ in range(3)]
      + [pltpu.SemaphoreType.DMA for _ in range(_NR)]
      + [pltpu.SemaphoreType.DMA for _ in range(_NR)]
  )

  @functools.partial(
      pl.kernel,
      out_type=jax.ShapeDtypeStruct((n_slabs, _NP, _FW), jnp.float32),
      mesh=mesh,
      scratch_types=scratch,
      compiler_params=pltpu.CompilerParams(use_tc_tiling_on_sc=False),
  )
  def k(table, src_h, dst_h, out, *scr):
    srcb = scr[0:3]
    dstb = scr[3:6]
    gib = scr[6:6 + _NI]
    rows = scr[6 + _NI:6 + _NI + _NR]
    zbuf = scr[6 + _NI + _NR]
    acc = scr[6 + _NI + _NR + 1]
    sem_i = scr[6 + _NI + _NR + 2:6 + _NI + _NR + 5]
    sem_g = scr[6 + _NI + _NR + 5:6 + _NI + 2 * _NR + 5]
    sem_s = scr[6 + _NI + 2 * _NR + 5:6 + _NI + 3 * _NR + 5]

    cid = lax.axis_index("c")
    sid = lax.axis_index("s")

    zero16 = jnp.zeros((16,), jnp.float32)

    def zfill(i, carry):
      zbuf[i] = zero16
      return carry

    lax.fori_loop(0, _RB, zfill, 0)

    for j in range(npp):
      slab = cid * npp + j

      def zrow(i, carry):
        pltpu.sync_copy(zbuf.at[:, pl.ds(0, _FW)],
                        acc.at[pl.ds(sid * rpt + i * _RB, _RB)])
        return carry

      lax.fori_loop(0, nrb, zrow, 0)
      plsc.subcore_barrier()

      if split_by_core:
        base_row = (cid * _NS + sid) * groups
      else:
        base_row = sid * groups

      def start_idx_block(t, bi):
        r0 = base_row + t * _NI
        pltpu.async_copy(src_h.at[pl.ds(r0, _NI)], srcb[bi], sem_i[bi])
        pltpu.async_copy(dst_h.at[pl.ds(r0, _NI)], dstb[bi], sem_i[bi])

      def wait_idx_block(bi):
        pltpu.make_async_copy(src_h.at[pl.ds(0, _NI)], srcb[bi],
                              sem_i[bi]).wait()
        pltpu.make_async_copy(dst_h.at[pl.ds(0, _NI)], dstb[bi],
                              sem_i[bi]).wait()

      def gidx_ref(bi, row, b8):
        if scale == 1:
          return srcb[bi].at[row]
        for kk in range(_C // 16):
          sl = pl.ds(kk * 16, 16)
          gib[b8][sl] = srcb[bi][row, sl] * scale + slab
        return gib[b8]

      def start_gather(bi, row, b8, b4):
        pltpu.async_copy(table.at[gidx_ref(bi, row, b8)], rows[b4],
                         sem_g[b4])

      def wait_gather(b4):
        pltpu.make_async_copy(table.at[dstb[0].at[0]], rows[b4],
                              sem_g[b4]).wait()

      def start_scatter(bi, row, b4):
        pltpu.async_copy(rows[b4], acc.at[dstb[bi].at[row]], sem_s[b4],
                         add=True)

      def wait_scatter(b4):
        pltpu.make_async_copy(rows[b4], acc.at[dstb[0].at[0]],
                              sem_s[b4]).wait()

      start_idx_block(0, 0)
      start_idx_block(1, 1)
      wait_idx_block(0)
      for b in range(2):
        start_gather(0, b, b, b)

      def body(t, bi_cur, bi_nxt, kind):
        for b in range(_NI):
          b4 = b % _NR
          a_b4 = (b + 2) % _NR
          a_b8 = (b + 2) % _NI
          do_a = (kind != 'last') or (b < _NI - 2)
          if do_a:
            if not (kind == 'first' and b < 2):
              wait_scatter(a_b4)
            if b == 6 and kind != 'last':
              wait_idx_block(bi_nxt)
            if b + 2 < _NI:
              start_gather(bi_cur, b + 2, a_b8, a_b4)
            else:
              start_gather(bi_nxt, b + 2 - _NI, a_b8, a_b4)
          wait_gather(b4)
          start_scatter(bi_cur, b, b4)
          if b == 2 and kind in ('first', 'mid'):
            start_idx_block(t + 2, (bi_cur + 2) % 3)

      body(0, 0, 1, 'first')
      n_mid = touter - 3
      m3 = n_mid // 3
      if m3 > 0:
        def mid(s, carry):
          ts = 1 + 3 * s
          for d in range(3):
            bi = (1 + d) % 3
            body(ts + d, bi, (bi + 1) % 3, 'mid')
          return carry
        lax.fori_loop(0, m3, mid, 0)
      for t in range(1 + 3 * m3, touter - 2):
        body(t, t % 3, (t + 1) % 3, 'mid')
      body(touter - 2, (touter - 2) % 3, (touter - 1) % 3, 'pen')
      body(touter - 1, (touter - 1) % 3, touter % 3, 'last')

      for b in range(_NR):
        wait_scatter(b)

      plsc.subcore_barrier()

      def rback(i, carry):
        r0 = sid * rpt + i * _RB
        pltpu.sync_copy(acc.at[pl.ds(r0, _RB)], out.at[slab, pl.ds(r0, _RB)])
        return carry

      lax.fori_loop(0, nrb, rback, 0)

  return k


def _tc1_body(xr, ar, wr, wl, br, out):
  a = ar[0] + ar[1]
  z = jnp.dot(xr[...], wr[...], preferred_element_type=jnp.float32)
  z = z + jnp.dot(a, wl[...], preferred_element_type=jnp.float32)
  z = z + br[...]
  out[...] = jnp.maximum(z, 0.0)


def _make_tc2_body(n_grid, n_graphs):
  def body(h1r, a2r, btr, w2r, w2l, b2r, l1w, l1b, l2w, l2b, outr, sums,
           counts):
    i = pl.program_id(0)

    @pl.when(i == 0)
    def _():
      sums[...] = jnp.zeros_like(sums)
      counts[...] = jnp.zeros_like(counts)

    z = jnp.dot(h1r[...], w2r[...], preferred_element_type=jnp.float32)
    for p in range(64 // _FW):
      z += jnp.dot(a2r[p], w2l[p],
                   preferred_element_type=jnp.float32)
    z = z + b2r[...]
    h2 = jnp.maximum(z, 0.0)

    iota_g = lax.broadcasted_iota(jnp.int32, (1, n_graphs), 1)
    for q in range(16):
      btq = btr[0, q, :]
      ohq = (btq[:, None] == iota_g).astype(jnp.float32)
      sums[...] += lax.dot_general(
          ohq, h2[:, q * 64:(q + 1) * 64], (((0,), (0,)), ((), ())),
          preferred_element_type=jnp.float32)
      counts[...] += jnp.sum(ohq, axis=0, keepdims=True)

    @pl.when(i == n_grid - 1)
    def _():
      cnt = jnp.maximum(counts[0, :], 1.0)
      pooled = sums[...] / cnt[:, None]
      h3 = jnp.maximum(
          jnp.dot(pooled, l1w[...], preferred_element_type=jnp.float32)
          + l1b[...], 0.0)
      outr[...] = jnp.dot(h3, l2w[...],
                          preferred_element_type=jnp.float32) + l2b[...]

  return body


def kernel(x, edge_index, batch, W1_root, W1_rel, b1, W2_root, W2_rel, b2,
           lin1_W, lin1_b, lin2_W, lin2_b):
  n, f = x.shape
  e = edge_index.shape[1]
  g = 128
  bn = 2048
  ng = _NP // bn

  unit = _NC * _NS * _C * _NI
  epad = ((e + unit - 1) // unit) * unit
  pad = epad - e
  src = jnp.concatenate([edge_index[0],
                         jnp.zeros((pad,), jnp.int32)]).reshape(-1, _C)
  dst = jnp.concatenate([edge_index[1],
                         jnp.full((pad,), n, jnp.int32)]).reshape(-1, _C)

  x8 = jnp.pad(x, ((0, _NP - n), (0, _FW - f)))
  w1r8 = jnp.pad(W1_root, ((0, _FW - f), (0, 0)))
  w1l8 = jnp.pad(W1_rel, ((0, _FW - f), (0, 0)))
  batch_p = jnp.concatenate([batch, jnp.full((_NP - n,), g, jnp.int32)])

  eye16 = jnp.eye(16, dtype=jnp.float32)
  w1r16 = jnp.kron(eye16, w1r8)
  w1l16 = jnp.kron(eye16, w1l8)
  b1l = jnp.tile(b1, 16).reshape(1, 1024)
  w2r16 = jnp.kron(eye16, W2_root)
  w2l16 = jnp.stack([jnp.kron(eye16, W2_rel[p * _FW:(p + 1) * _FW, :])
                     for p in range(64 // _FW)])
  b2l = jnp.tile(b2, 16).reshape(1, 1024)
  nr16 = _NP // 16
  batch16 = batch_p.reshape(ng, bn // 16, 16).transpose(0, 2, 1)

  agg1p = _make_sc_agg(epad, npp=1, split_by_core=True, scale=1)(
      x8, src, dst)

  h1l = pl.pallas_call(
      _tc1_body,
      grid=(ng,),
      in_specs=[
          pl.BlockSpec((bn // 16, 128), lambda i: (i, 0)),
          pl.BlockSpec((2, bn // 16, 128), lambda i: (0, i, 0)),
          pl.BlockSpec((128, 1024), lambda i: (0, 0)),
          pl.BlockSpec((128, 1024), lambda i: (0, 0)),
          pl.BlockSpec((1, 1024), lambda i: (0, 0)),
      ],
      out_specs=pl.BlockSpec((bn // 16, 1024), lambda i: (i, 0)),
      out_shape=jax.ShapeDtypeStruct((nr16, 1024), jnp.float32),
  )(x8.reshape(nr16, 128), agg1p.reshape(2, nr16, 128), w1r16, w1l16, b1l)

  npp2 = (64 // _FW) // _NC
  agg2 = _make_sc_agg(epad, npp=npp2, split_by_core=False, scale=64 // _FW)(
      h1l.reshape(_NP * (64 // _FW), _FW), src, dst)

  out = pl.pallas_call(
      _make_tc2_body(ng, g),
      grid=(ng,),
      in_specs=[
          pl.BlockSpec((bn // 16, 1024), lambda i: (i, 0)),
          pl.BlockSpec((64 // _FW, bn // 16, 128), lambda i: (0, i, 0)),
          pl.BlockSpec((1, 16, bn // 16), lambda i: (i, 0, 0)),
          pl.BlockSpec((1024, 1024), lambda i: (0, 0)),
          pl.BlockSpec((64 // _FW, 128, 1024), lambda i: (0, 0, 0)),
          pl.BlockSpec((1, 1024), lambda i: (0, 0)),
          pl.BlockSpec((64, 32), lambda i: (0, 0)),
          pl.BlockSpec((1, 32), lambda i: (0, 0)),
          pl.BlockSpec((32, 2), lambda i: (0, 0)),
          pl.BlockSpec((1, 2), lambda i: (0, 0)),
      ],
      out_specs=pl.BlockSpec((g, 2), lambda i: (0, 0)),
      out_shape=jax.ShapeDtypeStruct((g, 2), jnp.float32),
      scratch_shapes=[
          pltpu.VMEM((g, 64), jnp.float32),
          pltpu.VMEM((1, g), jnp.float32),
      ],
  )(h1l, agg2.reshape(64 // _FW, nr16, 128), batch16, w2r16, w2l16, b2l,
    lin1_W, lin1_b.reshape(1, 32), lin2_W, lin2_b.reshape(1, 2))

  return out


# speedup vs baseline: 12.4227x; 1.2011x over previous
import functools

import jax
import jax.numpy as jnp
from jax import lax
from jax.experimental import pallas as pl
from jax.experimental.pallas import tpu as pltpu
from jax.experimental.pallas import tpu_sc as plsc

_NC = 2
_NS = 16
_C = 128
_NP = 102400
_RB = 3200
_FW = 8
_NI = 8
_NR = 4


def _make_sc_agg(epad, npp, split_by_core, scale):
  n_slabs = _NC * npp
  workers = _NC * _NS if split_by_core else _NS
  ept = epad // workers
  groups = ept // _C
  assert groups % _NI == 0 and groups >= 4 * _NI
  touter = groups // _NI
  rpt = _NP // _NS
  nrb = rpt // _RB

  mesh = plsc.VectorSubcoreMesh(core_axis_name="c", subcore_axis_name="s")

  scratch = (
      [pltpu.VMEM((_NI, _C), jnp.int32) for _ in range(3)]
      + [pltpu.VMEM((_NI, _C), jnp.int32) for _ in range(3)]
      + [pltpu.VMEM((_C,), jnp.int32) for _ in range(_NI)]
      + [pltpu.VMEM((_C, _FW), jnp.float32) for _ in range(_NI)]
      + [
          pltpu.VMEM((_RB, 16), jnp.float32),
          pltpu.VMEM_SHARED((_NP, _FW), jnp.float32),
      ]
      + [pltpu.SemaphoreType.DMA for _ in range(3)]
      + [pltpu.SemaphoreType.DMA for _ in range(_NI)]
      + [pltpu.SemaphoreType.DMA for _ in range(_NI)]
  )

  @functools.partial(
      pl.kernel,
      out_type=jax.ShapeDtypeStruct((n_slabs, _NP, _FW), jnp.float32),
      mesh=mesh,
      scratch_types=scratch,
      compiler_params=pltpu.CompilerParams(use_tc_tiling_on_sc=False),
  )
  def k(table, src_h, dst_h, out, *scr):
    srcb = scr[0:3]
    dstb = scr[3:6]
    gib = scr[6:6 + _NI]
    rows = scr[6 + _NI:6 + 2 * _NI]
    zbuf = scr[6 + 2 * _NI]
    acc = scr[6 + 2 * _NI + 1]
    sem_i = scr[6 + 2 * _NI + 2:6 + 2 * _NI + 5]
    sem_g = scr[6 + 2 * _NI + 5:6 + 3 * _NI + 5]
    sem_s = scr[6 + 3 * _NI + 5:6 + 4 * _NI + 5]

    cid = lax.axis_index("c")
    sid = lax.axis_index("s")

    zero16 = jnp.zeros((16,), jnp.float32)

    def zfill(i, carry):
      zbuf[i] = zero16
      return carry

    lax.fori_loop(0, _RB, zfill, 0)

    for j in range(npp):
      slab = cid * npp + j

      def zrow(i, carry):
        pltpu.sync_copy(zbuf.at[:, pl.ds(0, _FW)],
                        acc.at[pl.ds(sid * rpt + i * _RB, _RB)])
        return carry

      lax.fori_loop(0, nrb, zrow, 0)
      plsc.subcore_barrier()

      if split_by_core:
        base_row = (cid * _NS + sid) * groups
      else:
        base_row = sid * groups

      def start_idx_block(t, bi):
        r0 = base_row + t * _NI
        pltpu.async_copy(src_h.at[pl.ds(r0, _NI)], srcb[bi], sem_i[bi])
        pltpu.async_copy(dst_h.at[pl.ds(r0, _NI)], dstb[bi], sem_i[bi])

      def wait_idx_block(bi):
        pltpu.make_async_copy(src_h.at[pl.ds(0, _NI)], srcb[bi],
                              sem_i[bi]).wait()
        pltpu.make_async_copy(dst_h.at[pl.ds(0, _NI)], dstb[bi],
                              sem_i[bi]).wait()

      def gidx_ref(bi, row, b8):
        if scale == 1:
          return srcb[bi].at[row]
        for kk in range(_C // 16):
          sl = pl.ds(kk * 16, 16)
          gib[b8][sl] = srcb[bi][row, sl] * scale + slab
        return gib[b8]

      def start_gather(bi, row, b8, b4):
        pltpu.async_copy(table.at[gidx_ref(bi, row, b8)], rows[b4],
                         sem_g[b4])

      def wait_gather(b4):
        pltpu.make_async_copy(table.at[dstb[0].at[0]], rows[b4],
                              sem_g[b4]).wait()

      def start_scatter(bi, row, b4):
        pltpu.async_copy(rows[b4], acc.at[dstb[bi].at[row]], sem_s[b4],
                         add=True)

      def wait_scatter(b4):
        pltpu.make_async_copy(rows[b4], acc.at[dstb[0].at[0]],
                              sem_s[b4]).wait()

      start_idx_block(0, 0)
      start_idx_block(1, 1)
      wait_idx_block(0)
      for b in range(4):
        start_gather(0, b, b, b)

      def body(t, bi_cur, bi_nxt, kind):
        for b in range(_NI):
          a_s = (b + 4) % _NI
          do_a = (kind != 'last') or (b < _NI - 4)
          if do_a:
            if not (kind == 'first' and b < 4):
              wait_scatter(a_s)
            if b == 3 and kind != 'last':
              wait_idx_block(bi_nxt)
            if b + 4 < _NI:
              start_gather(bi_cur, b + 4, a_s, a_s)
            else:
              start_gather(bi_nxt, b - 4, a_s, a_s)
          wait_gather(b)
          start_scatter(bi_cur, b, b)
          if b == 4 and kind in ('first', 'mid'):
            start_idx_block(t + 2, (bi_cur + 2) % 3)

      body(0, 0, 1, 'first')
      n_mid = touter - 3
      m3 = n_mid // 3
      if m3 > 0:
        def mid(s, carry):
          ts = 1 + 3 * s
          for d in range(3):
            bi = (1 + d) % 3
            body(ts + d, bi, (bi + 1) % 3, 'mid')
          return carry
        lax.fori_loop(0, m3, mid, 0)
      for t in range(1 + 3 * m3, touter - 2):
        body(t, t % 3, (t + 1) % 3, 'mid')
      body(touter - 2, (touter - 2) % 3, (touter - 1) % 3, 'pen')
      body(touter - 1, (touter - 1) % 3, touter % 3, 'last')

      for b in range(_NI):
        wait_scatter(b)

      plsc.subcore_barrier()

      def rback(i, carry):
        r0 = sid * rpt + i * _RB
        pltpu.sync_copy(acc.at[pl.ds(r0, _RB)], out.at[slab, pl.ds(r0, _RB)])
        return carry

      lax.fori_loop(0, nrb, rback, 0)

  return k


def _tc1_body(xr, ar, wr, wl, br, out):
  a = ar[0] + ar[1]
  z = jnp.dot(xr[...], wr[...], preferred_element_type=jnp.float32)
  z = z + jnp.dot(a, wl[...], preferred_element_type=jnp.float32)
  z = z + br[...]
  out[...] = jnp.maximum(z, 0.0)


def _make_tc2_body(n_grid, n_graphs):
  def body(h1r, a2r, btr, w2r, w2l, b2r, l1w, l1b, l2w, l2b, outr, sums,
           counts):
    i = pl.program_id(0)

    @pl.when(i == 0)
    def _():
      sums[...] = jnp.zeros_like(sums)
      counts[...] = jnp.zeros_like(counts)

    z = jnp.dot(h1r[...], w2r[...], preferred_element_type=jnp.float32)
    for p in range(64 // _FW):
      z += jnp.dot(a2r[p], w2l[p],
                   preferred_element_type=jnp.float32)
    z = z + b2r[...]
    h2 = jnp.maximum(z, 0.0)

    iota_g = lax.broadcasted_iota(jnp.int32, (1, n_graphs), 1)
    for q in range(16):
      btq = btr[0, q, :]
      ohq = (btq[:, None] == iota_g).astype(jnp.float32)
      sums[...] += lax.dot_general(
          ohq, h2[:, q * 64:(q + 1) * 64], (((0,), (0,)), ((), ())),
          preferred_element_type=jnp.float32)
      counts[...] += jnp.sum(ohq, axis=0, keepdims=True)

    @pl.when(i == n_grid - 1)
    def _():
      cnt = jnp.maximum(counts[0, :], 1.0)
      pooled = sums[...] / cnt[:, None]
      h3 = jnp.maximum(
          jnp.dot(pooled, l1w[...], preferred_element_type=jnp.float32)
          + l1b[...], 0.0)
      outr[...] = jnp.dot(h3, l2w[...],
                          preferred_element_type=jnp.float32) + l2b[...]

  return body


def kernel(x, edge_index, batch, W1_root, W1_rel, b1, W2_root, W2_rel, b2,
           lin1_W, lin1_b, lin2_W, lin2_b):
  n, f = x.shape
  e = edge_index.shape[1]
  g = 128
  bn = 2048
  ng = _NP // bn

  unit = _NC * _NS * _C * _NI
  epad = ((e + unit - 1) // unit) * unit
  pad = epad - e
  src = jnp.concatenate([edge_index[0],
                         jnp.zeros((pad,), jnp.int32)]).reshape(-1, _C)
  dst = jnp.concatenate([edge_index[1],
                         jnp.full((pad,), n, jnp.int32)]).reshape(-1, _C)

  x8 = jnp.pad(x, ((0, _NP - n), (0, _FW - f)))
  w1r8 = jnp.pad(W1_root, ((0, _FW - f), (0, 0)))
  w1l8 = jnp.pad(W1_rel, ((0, _FW - f), (0, 0)))
  batch_p = jnp.concatenate([batch, jnp.full((_NP - n,), g, jnp.int32)])

  eye16 = jnp.eye(16, dtype=jnp.float32)
  w1r16 = jnp.kron(eye16, w1r8)
  w1l16 = jnp.kron(eye16, w1l8)
  b1l = jnp.tile(b1, 16).reshape(1, 1024)
  w2r16 = jnp.kron(eye16, W2_root)
  w2l16 = jnp.stack([jnp.kron(eye16, W2_rel[p * _FW:(p + 1) * _FW, :])
                     for p in range(64 // _FW)])
  b2l = jnp.tile(b2, 16).reshape(1, 1024)
  nr16 = _NP // 16
  batch16 = batch_p.reshape(ng, bn // 16, 16).transpose(0, 2, 1)

  agg1p = _make_sc_agg(epad, npp=1, split_by_core=True, scale=1)(
      x8, src, dst)

  h1l = pl.pallas_call(
      _tc1_body,
      grid=(ng,),
      in_specs=[
          pl.BlockSpec((bn // 16, 128), lambda i: (i, 0)),
          pl.BlockSpec((2, bn // 16, 128), lambda i: (0, i, 0)),
          pl.BlockSpec((128, 1024), lambda i: (0, 0)),
          pl.BlockSpec((128, 1024), lambda i: (0, 0)),
          pl.BlockSpec((1, 1024), lambda i: (0, 0)),
      ],
      out_specs=pl.BlockSpec((bn // 16, 1024), lambda i: (i, 0)),
      out_shape=jax.ShapeDtypeStruct((nr16, 1024), jnp.float32),
  )(x8.reshape(nr16, 128), agg1p.reshape(2, nr16, 128), w1r16, w1l16, b1l)

  npp2 = (64 // _FW) // _NC
  agg2 = _make_sc_agg(epad, npp=npp2, split_by_core=False, scale=64 // _FW)(
      h1l.reshape(_NP * (64 // _FW), _FW), src, dst)

  out = pl.pallas_call(
      _make_tc2_body(ng, g),
      grid=(ng,),
      in_specs=[
          pl.BlockSpec((bn // 16, 1024), lambda i: (i, 0)),
          pl.BlockSpec((64 // _FW, bn // 16, 128), lambda i: (0, i, 0)),
          pl.BlockSpec((1, 16, bn // 16), lambda i: (i, 0, 0)),
          pl.BlockSpec((1024, 1024), lambda i: (0, 0)),
          pl.BlockSpec((64 // _FW, 128, 1024), lambda i: (0, 0, 0)),
          pl.BlockSpec((1, 1024), lambda i: (0, 0)),
          pl.BlockSpec((64, 32), lambda i: (0, 0)),
          pl.BlockSpec((1, 32), lambda i: (0, 0)),
          pl.BlockSpec((32, 2), lambda i: (0, 0)),
          pl.BlockSpec((1, 2), lambda i: (0, 0)),
      ],
      out_specs=pl.BlockSpec((g, 2), lambda i: (0, 0)),
      out_shape=jax.ShapeDtypeStruct((g, 2), jnp.float32),
      scratch_shapes=[
          pltpu.VMEM((g, 64), jnp.float32),
          pltpu.VMEM((1, g), jnp.float32),
      ],
  )(h1l, agg2.reshape(64 // _FW, nr16, 128), batch16, w2r16, w2l16, b2l,
    lin1_W, lin1_b.reshape(1, 32), lin2_W, lin2_b.reshape(1, 2))

  return out
